# Initial kernel scaffold; baseline (speedup 1.0000x reference)
import jax
import jax.numpy as jnp
from jax.experimental import pallas as pl


def kernel(input, adj, W, mlp_w, mlp_b):
    raise NotImplementedError("write your pallas kernel here")


# speedup vs baseline: 1.1531x; 1.1531x over previous
import functools

import jax
import jax.numpy as jnp
from jax.experimental import pallas as pl
from jax.experimental.pallas import tpu as pltpu


def _body(x_ref, adj_ref, w_ref, mlp_w_ref, mlp_b_ref, out_ref, xw_ref):
    @pl.when(pl.program_id(0) == 0)
    def _():
        xw_ref[...] = jnp.dot(x_ref[...], w_ref[...],
                              preferred_element_type=jnp.float32)

    a = jnp.dot(adj_ref[...], xw_ref[...], preferred_element_type=jnp.float32)
    a = jnp.maximum(a, 0.0)
    y = jax.lax.dot_general(a, mlp_w_ref[...],
                            dimension_numbers=(((1,), (1,)), ((), ())),
                            preferred_element_type=jnp.float32)
    y = y + mlp_b_ref[...]
    zmax = jnp.max(y, axis=1, keepdims=True)
    zmin = jnp.min(y, axis=1, keepdims=True)
    z = (y - zmin) / (zmax - zmin)
    nrm = jnp.sqrt(jnp.sum(z * z, axis=1, keepdims=True))
    out_ref[...] = z / jnp.maximum(nrm, 1e-12)


@functools.partial(jax.jit, static_argnames=("bm",))
def _run(x, adj, W, mlp_w, mlp_b2, bm):
    n, d_in = x.shape
    d_hid = W.shape[1]
    d_out = mlp_w.shape[0]
    grid = (n // bm,)
    return pl.pallas_call(
        _body,
        grid=grid,
        in_specs=[
            pl.BlockSpec((n, d_in), lambda i: (0, 0)),
            pl.BlockSpec((bm, n), lambda i: (i, 0)),
            pl.BlockSpec((d_in, d_hid), lambda i: (0, 0)),
            pl.BlockSpec((d_out, d_hid), lambda i: (0, 0)),
            pl.BlockSpec((1, d_out), lambda i: (0, 0)),
        ],
        out_specs=pl.BlockSpec((bm, d_out), lambda i: (i, 0)),
        out_shape=jax.ShapeDtypeStruct((n, d_out), jnp.float32),
        scratch_shapes=[pltpu.VMEM((n, d_hid), jnp.float32)],
        compiler_params=pltpu.CompilerParams(
            dimension_semantics=("arbitrary",),
        ),
    )(x, adj, W, mlp_w, mlp_b2)


def kernel(input, adj, W, mlp_w, mlp_b):
    n = input.shape[0]
    bm = next((b for b in (400, 200, 80, 40, 8, 1) if n % b == 0))
    return _run(input, adj, W, mlp_w, mlp_b.reshape(1, -1), bm)


# speedup vs baseline: 1.1553x; 1.0019x over previous
import functools

import jax
import jax.numpy as jnp
from jax.experimental import pallas as pl
from jax.experimental.pallas import tpu as pltpu


def _body(x_ref, adj_ref, w_ref, mlp_w_ref, mlp_b_ref, out_ref, xw_ref):
    @pl.when(pl.program_id(0) == 0)
    def _():
        xw_ref[...] = jnp.dot(x_ref[...], w_ref[...],
                              preferred_element_type=jnp.float32
                              ).astype(jnp.bfloat16)

    a = jnp.dot(adj_ref[...].astype(jnp.bfloat16), xw_ref[...],
                preferred_element_type=jnp.float32)
    a = jnp.maximum(a, 0.0)
    y = jax.lax.dot_general(a, mlp_w_ref[...],
                            dimension_numbers=(((1,), (1,)), ((), ())),
                            preferred_element_type=jnp.float32)
    y = y + mlp_b_ref[...]
    zmax = jnp.max(y, axis=1, keepdims=True)
    zmin = jnp.min(y, axis=1, keepdims=True)
    z = (y - zmin) / (zmax - zmin)
    nrm = jnp.sqrt(jnp.sum(z * z, axis=1, keepdims=True))
    out_ref[...] = z / jnp.maximum(nrm, 1e-12)


@functools.partial(jax.jit, static_argnames=("bm",))
def _run(x, adj, W, mlp_w, mlp_b2, bm):
    n, d_in = x.shape
    d_hid = W.shape[1]
    d_out = mlp_w.shape[0]
    grid = (n // bm,)
    return pl.pallas_call(
        _body,
        grid=grid,
        in_specs=[
            pl.BlockSpec((n, d_in), lambda i: (0, 0)),
            pl.BlockSpec((bm, n), lambda i: (i, 0)),
            pl.BlockSpec((d_in, d_hid), lambda i: (0, 0)),
            pl.BlockSpec((d_out, d_hid), lambda i: (0, 0)),
            pl.BlockSpec((1, d_out), lambda i: (0, 0)),
        ],
        out_specs=pl.BlockSpec((bm, d_out), lambda i: (i, 0)),
        out_shape=jax.ShapeDtypeStruct((n, d_out), jnp.float32),
        scratch_shapes=[pltpu.VMEM((n, d_hid), jnp.bfloat16)],
        compiler_params=pltpu.CompilerParams(
            dimension_semantics=("arbitrary",),
        ),
    )(x, adj, W, mlp_w, mlp_b2)


def kernel(input, adj, W, mlp_w, mlp_b):
    n = input.shape[0]
    bm = next((b for b in (400, 200, 80, 40, 8, 1) if n % b == 0))
    return _run(input, adj, W, mlp_w, mlp_b.reshape(1, -1), bm)
